# baseline (speedup 1.0000x reference)
import jax
import jax.numpy as jnp
from jax.experimental import pallas as pl
from jax.experimental.pallas import tpu as pltpu

N = 256
CH = 32
PACK = 4
G = 8


def _leaky(x):
    return jnp.maximum(x, 0.01 * x)


def _gnn_block(*refs):
    ns = (len(refs) - 18) // 2
    v_refs = refs[:ns]
    (w1, bb1, w2, bb2, w3ab, bb3, w4, bb4,
     w5, bb5, w6, bb6, w7, bb7, w8, bb8, s2, bgr) = refs[ns:ns + 18]
    out_refs = refs[ns + 18:]
    R = N * G

    def dotf(a, w):
        return jax.lax.dot_general(a, w, (((1,), (0,)), ((), ())),
                                   preferred_element_type=jnp.float32)

    def dot(a, w):
        return dotf(a, w).astype(jnp.bfloat16)

    def roll_up(a):
        return jnp.concatenate([a[G:], a[:G]], axis=0)

    def roll_dn(a):
        return jnp.concatenate([a[R - G:], a[:R - G]], axis=0)

    def lin(w, bb):
        return lambda x: _leaky(dot(x, w[...]) + bb[...])

    def per(f, xs):
        return [f(x) for x in xs]

    def msgs(xs):
        def one(f):
            c = dot(f, w3ab[...])
            return _leaky(c[:, :PACK * CH] + roll_up(c[:, PACK * CH:])
                          + bb3[...])
        ms = per(one, xs)
        return per(lin(w4, bb4), ms)

    xs = [v[...].reshape(R, PACK * 11) for v in v_refs]
    xs = per(lin(w1, bb1), xs)
    xs = per(lin(w2, bb2), xs)

    ms = msgs(xs)
    nvs = per(lambda m: m + roll_dn(m), ms)
    for _ in range(5):
        hs = per(lin(w5, bb5), nvs)
        hs = per(lin(w6, bb6), hs)
        ms = msgs(hs)
        nvs = [nv + (m + roll_dn(m)) for nv, m in zip(nvs, ms)]

    fs = per(lin(w7, bb7), nvs)
    fs = per(lin(w8, bb8), fs)
    ss = per(lambda f: dotf(s2[...], f), fs)
    for o_ref, s in zip(out_refs, ss):
        o_ref[...] = jax.nn.sigmoid(s + bgr[...])


def kernel(vertices, W1, b1, W2, b2, W3, b3, W4, b4, W5, b5, W6, b6,
           W7, b7, W8, b8, Wg, bg, edges, dest_edges):
    B = vertices.shape[0]
    f32 = jnp.float32
    ngroups = B // PACK
    S = 4
    nblocks = ngroups // (S * G)
    R = N * G

    vt = jnp.transpose(vertices.astype(jnp.bfloat16),
                       (1, 0, 2)).reshape(N, ngroups, PACK * 11)

    eye = jnp.eye(PACK, dtype=f32)

    def bd(w):
        return jnp.kron(eye, w.T.astype(f32)).astype(jnp.bfloat16)

    def bt(b):
        return jnp.tile(b.astype(f32), PACK)[None, :].astype(jnp.bfloat16)

    w1 = bd(W1)
    w2, w5, w6, w7 = bd(W2), bd(W5), bd(W6), bd(W7)
    w4 = bd(W4) * 0.5
    w3ab = jnp.concatenate([bd(W3[:, :CH]), bd(W3[:, CH:])], axis=1)
    w8 = bd(W8)
    bb1, bb2, bb3, bb4 = bt(b1), bt(b2), bt(b3), bt(b4) * 0.5
    bb5, bb6, bb7, bb8 = bt(b5), bt(b6), bt(b7), bt(b8)

    r = jnp.arange(R)
    s2 = (((r[None, :] % G) == jnp.arange(G)[:, None]).astype(f32)
          * Wg[0, r // G][None, :].astype(f32)).astype(jnp.bfloat16)
    bgr = jnp.broadcast_to(bg.astype(f32).reshape(1, 1), (1, 1))

    full = lambda a: pl.BlockSpec(a.shape, lambda i: (0,) * a.ndim)
    ws = [w1, bb1, w2, bb2, w3ab, bb3, w4, bb4,
          w5, bb5, w6, bb6, w7, bb7, w8, bb8, s2, bgr]

    def vspec(off):
        return pl.BlockSpec((N, G, PACK * 11),
                            lambda i: (0, S * i + off, 0))
    ospec = pl.BlockSpec((G, PACK), lambda i: (i, 0))
    outs = pl.pallas_call(
        _gnn_block,
        grid=(nblocks,),
        in_specs=[vspec(o) for o in range(S)] + [full(a) for a in ws],
        out_specs=(ospec,) * S,
        out_shape=(jax.ShapeDtypeStruct((nblocks * G, PACK), f32),) * S,
        compiler_params=pltpu.CompilerParams(
            dimension_semantics=("parallel",)),
    )(*([vt] * S), *ws)
    out = jnp.concatenate([o.reshape(nblocks, G, PACK) for o in outs], axis=1)
    return out.reshape(B, 1)


# speedup vs baseline: 1.0004x; 1.0004x over previous
import jax
import jax.numpy as jnp
from jax.experimental import pallas as pl
from jax.experimental.pallas import tpu as pltpu

N = 256
CH = 32
PACK = 4
G = 8


def _leaky(x):
    return jnp.maximum(x, 0.01 * x)


def _gnn_block(*refs):
    ns = (len(refs) - 19) // 2
    v_refs = refs[:ns]
    (w1, bb1, w2, bb2, w3a, w3b, bb3, w4, bb4,
     w5, bb5, w6, bb6, w7, bb7, w8, bb8, s2, bgr) = refs[ns:ns + 19]
    out_refs = refs[ns + 19:]
    R = N * G

    def dotf(a, w):
        return jax.lax.dot_general(a, w, (((1,), (0,)), ((), ())),
                                   preferred_element_type=jnp.float32)

    def dot(a, w):
        return dotf(a, w).astype(jnp.bfloat16)

    def roll_up(a):
        return jnp.concatenate([a[G:], a[:G]], axis=0)

    def roll_dn(a):
        return jnp.concatenate([a[R - G:], a[:R - G]], axis=0)

    def lin(w, bb):
        return lambda x: _leaky(dot(x, w[...]) + bb[...])

    def per(f, xs):
        return [f(x) for x in xs]

    def msgs(xs):
        ms = per(lambda f: _leaky(dot(f, w3a[...])
                                  + roll_up(dot(f, w3b[...])) + bb3[...]), xs)
        return per(lin(w4, bb4), ms)

    xs = [v[...].reshape(R, PACK * 11) for v in v_refs]
    xs = per(lin(w1, bb1), xs)
    xs = per(lin(w2, bb2), xs)

    ms = msgs(xs)
    nvs = per(lambda m: m + roll_dn(m), ms)
    for _ in range(5):
        hs = per(lin(w5, bb5), nvs)
        hs = per(lin(w6, bb6), hs)
        ms = msgs(hs)
        nvs = [nv + (m + roll_dn(m)) for nv, m in zip(nvs, ms)]

    fs = per(lin(w7, bb7), nvs)
    fs = per(lin(w8, bb8), fs)
    ss = per(lambda f: dotf(s2[...], f), fs)
    for o_ref, s in zip(out_refs, ss):
        o_ref[...] = jax.nn.sigmoid(s + bgr[...])


def kernel(vertices, W1, b1, W2, b2, W3, b3, W4, b4, W5, b5, W6, b6,
           W7, b7, W8, b8, Wg, bg, edges, dest_edges):
    B = vertices.shape[0]
    f32 = jnp.float32
    ngroups = B // PACK
    S = 4
    nblocks = ngroups // (S * G)
    R = N * G

    vt = jnp.transpose(vertices.astype(jnp.bfloat16),
                       (1, 0, 2)).reshape(N, ngroups, PACK * 11)

    eye = jnp.eye(PACK, dtype=f32)

    def bd(w):
        return jnp.kron(eye, w.T.astype(f32)).astype(jnp.bfloat16)

    def bt(b):
        return jnp.tile(b.astype(f32), PACK)[None, :].astype(jnp.bfloat16)

    w1 = bd(W1)
    w2, w5, w6, w7 = bd(W2), bd(W5), bd(W6), bd(W7)
    w4 = bd(W4) * 0.5
    w3a, w3b = bd(W3[:, :CH]), bd(W3[:, CH:])
    w8 = bd(W8)
    bb1, bb2, bb3, bb4 = bt(b1), bt(b2), bt(b3), bt(b4) * 0.5
    bb5, bb6, bb7, bb8 = bt(b5), bt(b6), bt(b7), bt(b8)

    r = jnp.arange(R)
    s2 = (((r[None, :] % G) == jnp.arange(G)[:, None]).astype(f32)
          * Wg[0, r // G][None, :].astype(f32)).astype(jnp.bfloat16)
    bgr = jnp.broadcast_to(bg.astype(f32).reshape(1, 1), (1, 1))

    full = lambda a: pl.BlockSpec(a.shape, lambda i: (0,) * a.ndim)
    ws = [w1, bb1, w2, bb2, w3a, w3b, bb3, w4, bb4,
          w5, bb5, w6, bb6, w7, bb7, w8, bb8, s2, bgr]

    def vspec(off):
        return pl.BlockSpec((N, G, PACK * 11),
                            lambda i: (0, S * i + off, 0))
    ospec = pl.BlockSpec((G, PACK), lambda i: (i, 0))
    outs = pl.pallas_call(
        _gnn_block,
        grid=(nblocks,),
        in_specs=[vspec(o) for o in range(S)] + [full(a) for a in ws],
        out_specs=(ospec,) * S,
        out_shape=(jax.ShapeDtypeStruct((nblocks * G, PACK), f32),) * S,
        compiler_params=pltpu.CompilerParams(
            dimension_semantics=("parallel",)),
    )(*([vt] * S), *ws)
    out = jnp.concatenate([o.reshape(nblocks, G, PACK) for o in outs], axis=1)
    return out.reshape(B, 1)
